# Initial kernel scaffold; baseline (speedup 1.0000x reference)
import jax
import jax.numpy as jnp
from jax.experimental import pallas as pl


def kernel(x, edge_index, batch, known_mask, unk_mask, obs_mask, msg_weights, emb_W, emb_b, gcn_W, gcn_b, fc1_W, fc1_b, fc2_W, fc2_b):
    raise NotImplementedError("write your pallas kernel here")


# speedup vs baseline: 16.4822x; 16.4822x over previous
import functools
import jax
import jax.numpy as jnp
from jax import lax
from jax.experimental import pallas as pl
from jax.experimental.pallas import tpu as pltpu
from jax.experimental.pallas import tpu_sc as plsc

N = 100000
E = 1600000
G = 64
NF = 128
NH = 32

NC = 2
NS = 16
L = 16

BB = 512

EPAD = 1605632
NPAD = 100352
EH = EPAD // 2
NHALF = NPAD // 2
MPAD = 540672
NBLK_MASK = MPAD // (NS * BB)
NBLK_DEG = EH // (NS * BB)
NBLK_EDGE = EPAD // (NS * BB)


_DN = lax.GatherDimensionNumbers(offset_dims=(), collapsed_slice_dims=(0,),
                                 start_index_map=(0,))


def _splat(vec16, j):
  idx = jnp.full((L,), 0, jnp.int32) + j
  return lax.gather(vec16, idx[:, None], _DN, (1,),
                    mode=lax.GatherScatterMode.PROMISE_IN_BOUNDS)


def _fill_vmem(buf, n, val16):
  def body(i, _):
    buf[pl.ds(i * L, L)] = val16
    return 0
  lax.fori_loop(0, n // L, body, 0, unroll=8)


def _fill_spmem_1d(sp_ref, buf, start, count, val16):
  _fill_vmem(buf, 1024, val16)
  nfull = count // 1024
  rem = count % 1024
  def cp(i, _):
    pltpu.sync_copy(buf, sp_ref.at[pl.ds(start + i * 1024, 1024)])
    return 0
  lax.fori_loop(0, nfull, cp, 0)
  if rem:
    pltpu.sync_copy(buf.at[pl.ds(0, rem)],
                    sp_ref.at[pl.ds(start + nfull * 1024, rem)])


def _ew_body(km, um, om, spv, ew_out, ewsp, mi_v, idx_v, val_v, fbuf, spv_v):
  c = lax.axis_index("c")
  s = lax.axis_index("s")
  lo = c * EH
  pltpu.sync_copy(spv, spv_v)
  ones16 = jnp.ones((L,), jnp.float32)
  _fill_spmem_1d(ewsp, fbuf, s * (EH // NS), EH // NS, ones16)
  plsc.subcore_barrier()
  @pl.when(jnp.logical_and(c == 1, s == NS - 1))
  def _():
    _fill_spmem_1d(ewsp, fbuf, E - EH, EPAD - E,
                   jnp.zeros((L,), jnp.float32))
  plsc.subcore_barrier()

  masks = (km, um, om)
  for k in range(3):
    mref = masks[k]
    spk = _splat(spv_v[...], k)
    _fill_vmem(val_v, BB, spk)

    def blk(b, _):
      off = (b * NS + s) * BB
      pltpu.sync_copy(mref.at[pl.ds(off, BB)], mi_v)
      for i in range(BB // L):
        mi16 = mi_v[pl.ds(i * L, L)]
        inr = jnp.logical_and(mi16 >= lo, mi16 < lo + EH)
        loc = jnp.where(inr, mi16 - lo, EH)
        idx_v[pl.ds(i * L, L)] = loc
      pltpu.sync_copy(val_v, ewsp.at[idx_v])
      return 0

    lax.fori_loop(0, NBLK_MASK, blk, 0)
    plsc.subcore_barrier()

  t = EH // NS
  pltpu.sync_copy(ewsp.at[pl.ds(s * t, t)], ew_out.at[pl.ds(lo + s * t, t)])


def _ew_call(km, um, om, sp16):
  mesh = plsc.VectorSubcoreMesh(core_axis_name="c", subcore_axis_name="s")
  f = pl.kernel(
      _ew_body,
      out_type=jax.ShapeDtypeStruct((EPAD,), jnp.float32),
      mesh=mesh,
      scratch_types=[
          pltpu.VMEM_SHARED((EH + 8,), jnp.float32),
          pltpu.VMEM((BB,), jnp.int32),
          pltpu.VMEM((BB,), jnp.int32),
          pltpu.VMEM((BB,), jnp.float32),
          pltpu.VMEM((1024,), jnp.float32),
          pltpu.VMEM((L,), jnp.float32),
      ],
      compiler_params=pltpu.CompilerParams(use_tc_tiling_on_sc=False),
  )
  return f(km, um, om, sp16)


def _deg_body(col, ew, degp, degsp, ci_v, ev_v, fbuf):
  c = lax.axis_index("c")
  s = lax.axis_index("s")
  init = jnp.where(c == 0, 1.0, 0.0) * jnp.ones((L,), jnp.float32)
  _fill_spmem_1d(degsp, fbuf, s * (NPAD // NS), NPAD // NS, init)
  plsc.subcore_barrier()

  def blk(b, _):
    off = c * EH + (b * NS + s) * BB
    pltpu.sync_copy(col.at[pl.ds(off, BB)], ci_v)
    pltpu.sync_copy(ew.at[pl.ds(off, BB)], ev_v)
    pltpu.sync_copy(ev_v, degsp.at[ci_v], add=True)
    return 0

  lax.fori_loop(0, NBLK_DEG, blk, 0)
  plsc.subcore_barrier()
  t = NPAD // NS
  pltpu.sync_copy(degsp.at[pl.ds(s * t, t)], degp.at[c, pl.ds(s * t, t)])


def _deg_call(col, ew):
  mesh = plsc.VectorSubcoreMesh(core_axis_name="c", subcore_axis_name="s")
  f = pl.kernel(
      _deg_body,
      out_type=jax.ShapeDtypeStruct((NC, NPAD), jnp.float32),
      mesh=mesh,
      scratch_types=[
          pltpu.VMEM_SHARED((NPAD + 8,), jnp.float32),
          pltpu.VMEM((BB,), jnp.int32),
          pltpu.VMEM((BB,), jnp.float32),
          pltpu.VMEM((1024,), jnp.float32),
      ],
      compiler_params=pltpu.CompilerParams(use_tc_tiling_on_sc=False),
  )
  return f(col, ew)


def _edge_body(row, col, ew, hws, acc, accsp, ri_v, ci_v, ev_v, ewm_v,
               sidx_v, rows_v, zb, sem):
  c = lax.axis_index("c")
  s = lax.axis_index("s")
  lo = c * NHALF
  zero16 = jnp.zeros((L,), jnp.float32)
  def zv(i, _):
    zb[i, pl.ds(0, L)] = zero16
    zb[i, pl.ds(L, L)] = zero16
    return 0
  lax.fori_loop(0, 64, zv, 0, unroll=8)
  rows_per_tile = NHALF // NS
  r0 = s * rows_per_tile
  def zr(i, _):
    pltpu.sync_copy(zb, accsp.at[pl.ds(r0 + i * 64, 64), :])
    return 0
  lax.fori_loop(0, rows_per_tile // 64, zr, 0)
  @pl.when(s == 0)
  def _():
    pltpu.sync_copy(zb.at[pl.ds(0, 8), :], accsp.at[pl.ds(NHALF, 8), :])
  plsc.subcore_barrier()

  def blk(b, _):
    off = (b * NS + s) * BB
    pltpu.sync_copy(row.at[pl.ds(off, BB)], ri_v)
    pltpu.sync_copy(col.at[pl.ds(off, BB)], ci_v)
    pltpu.sync_copy(ew.at[pl.ds(off, BB)], ev_v)
    pltpu.async_copy(hws.at[ri_v], rows_v, sem).wait()
    for i in range(BB // L):
      ci16 = ci_v[pl.ds(i * L, L)]
      inr = jnp.logical_and(ci16 >= lo, ci16 < lo + NHALF)
      ewm_v[pl.ds(i * L, L)] = jnp.where(inr, ev_v[pl.ds(i * L, L)], 0.0)
      sidx_v[pl.ds(i * L, L)] = jnp.where(inr, ci16 - lo, NHALF)

    def sc(i, _):
      ewm16 = ewm_v[pl.ds(i * L, L)]
      for j in range(L):
        spl = _splat(ewm16, j)
        r = i * L + j
        rows_v[r, pl.ds(0, L)] = rows_v[r, pl.ds(0, L)] * spl
        rows_v[r, pl.ds(L, L)] = rows_v[r, pl.ds(L, L)] * spl
      return 0

    lax.fori_loop(0, BB // L, sc, 0, unroll=2)
    pltpu.sync_copy(rows_v, accsp.at[sidx_v], add=True)
    return 0

  lax.fori_loop(0, NBLK_EDGE, blk, 0)
  plsc.subcore_barrier()
  rpt = NHALF // NS
  pltpu.sync_copy(accsp.at[pl.ds(s * rpt, rpt), :],
                  acc.at[pl.ds(lo + s * rpt, rpt), :])


def _edge_call(row, col, ew, hws):
  mesh = plsc.VectorSubcoreMesh(core_axis_name="c", subcore_axis_name="s")
  f = pl.kernel(
      _edge_body,
      out_type=jax.ShapeDtypeStruct((NPAD, NH), jnp.float32),
      mesh=mesh,
      scratch_types=[
          pltpu.VMEM_SHARED((NHALF + 8, NH), jnp.float32),
          pltpu.VMEM((BB,), jnp.int32),
          pltpu.VMEM((BB,), jnp.int32),
          pltpu.VMEM((BB,), jnp.float32),
          pltpu.VMEM((BB,), jnp.float32),
          pltpu.VMEM((BB,), jnp.int32),
          pltpu.VMEM((BB, NH), jnp.float32),
          pltpu.VMEM((64, NH), jnp.float32),
          pltpu.SemaphoreType.DMA,
      ],
      compiler_params=pltpu.CompilerParams(use_tc_tiling_on_sc=False),
  )
  return f(row, col, ew, hws)


def _prep_body(mw_ref, embW_ref, embb_ref, gcnW_ref, sp_ref, wc_ref, bc_ref):
  mw = mw_ref[0, :]
  m = jnp.max(mw)
  e = jnp.exp(mw - m)
  sp_ref[0, :] = e / jnp.sum(e)
  w1 = gcnW_ref[1:, :]
  wc = jnp.dot(embW_ref[...], w1, preferred_element_type=jnp.float32)
  wc_ref[...] = jnp.concatenate([gcnW_ref[0:1, :], wc], axis=0)
  bc_ref[...] = jnp.dot(embb_ref[...], w1, preferred_element_type=jnp.float32)


def _prep_call(mw, embW, embb, gcnW):
  return pl.pallas_call(
      _prep_body,
      out_shape=(
          jax.ShapeDtypeStruct((1, 3), jnp.float32),
          jax.ShapeDtypeStruct((NF, NH), jnp.float32),
          jax.ShapeDtypeStruct((1, NH), jnp.float32),
      ),
  )(mw.reshape(1, 3), embW, embb.reshape(1, 63), gcnW)


def _mm_body(x_ref, wc_ref, bc_ref, out_ref):
  out_ref[...] = jnp.dot(x_ref[...], wc_ref[...],
                         preferred_element_type=jnp.float32) + bc_ref[...]


def _mm_call(xpad, wc, bc):
  nb = NPAD // 1024
  return pl.pallas_call(
      _mm_body,
      grid=(nb,),
      in_specs=[
          pl.BlockSpec((1024, NF), lambda i: (i, 0)),
          pl.BlockSpec((NF, NH), lambda i: (0, 0)),
          pl.BlockSpec((1, NH), lambda i: (0, 0)),
      ],
      out_specs=pl.BlockSpec((1024, NH), lambda i: (i, 0)),
      out_shape=jax.ShapeDtypeStruct((NPAD, NH), jnp.float32),
  )(xpad, wc, bc)


def _scale_body(d0_ref, d1_ref, hw_ref, hws_ref, dis_ref):
  deg = d0_ref[...] + d1_ref[...]
  dis = 1.0 / jnp.sqrt(deg)
  dis_ref[...] = dis
  hws_ref[...] = hw_ref[...] * dis


def _scale_call(d0, d1, hw):
  nb = NPAD // 1024
  return pl.pallas_call(
      _scale_body,
      grid=(nb,),
      in_specs=[
          pl.BlockSpec((1024, 1), lambda i: (i, 0)),
          pl.BlockSpec((1024, 1), lambda i: (i, 0)),
          pl.BlockSpec((1024, NH), lambda i: (i, 0)),
      ],
      out_specs=(
          pl.BlockSpec((1024, NH), lambda i: (i, 0)),
          pl.BlockSpec((1024, 1), lambda i: (i, 0)),
      ),
      out_shape=(
          jax.ShapeDtypeStruct((NPAD, NH), jnp.float32),
          jax.ShapeDtypeStruct((NPAD, 1), jnp.float32),
      ),
  )(d0, d1, hw)


def _final_body(acc_ref, hws_ref, dis_ref, batch_ref, gb_ref,
                f1w_ref, f1b_ref, f2w_ref, f2b_ref, out_ref, pool_ref):
  i = pl.program_id(0)
  @pl.when(i == 0)
  def _():
    pool_ref[...] = jnp.zeros_like(pool_ref)
  out = jax.nn.relu(dis_ref[...] * (acc_ref[...] + hws_ref[...]) + gb_ref[...])
  rid = i * 1024 + lax.broadcasted_iota(jnp.int32, (1024, NH), 0)
  out = jnp.where(rid < N, out, 0.0)
  oh = (batch_ref[...] == lax.broadcasted_iota(jnp.int32, (1024, G), 1))
  oh = oh.astype(jnp.float32)
  pool_ref[...] += lax.dot_general(oh, out, (((0,), (0,)), ((), ())),
                                   preferred_element_type=jnp.float32)
  z = jax.nn.relu(jnp.dot(pool_ref[...], f1w_ref[...],
                          preferred_element_type=jnp.float32) + f1b_ref[...])
  out_ref[...] = (jnp.dot(z, f2w_ref[...],
                          preferred_element_type=jnp.float32) + f2b_ref[...])


def _final_call(acc, hws, dis, batchp, gcn_b, f1w, f1b, f2w, f2b):
  nb = NPAD // 1024
  return pl.pallas_call(
      _final_body,
      grid=(nb,),
      in_specs=[
          pl.BlockSpec((1024, NH), lambda i: (i, 0)),
          pl.BlockSpec((1024, NH), lambda i: (i, 0)),
          pl.BlockSpec((1024, 1), lambda i: (i, 0)),
          pl.BlockSpec((1024, 1), lambda i: (i, 0)),
          pl.BlockSpec((1, NH), lambda i: (0, 0)),
          pl.BlockSpec((NH, NH), lambda i: (0, 0)),
          pl.BlockSpec((1, NH), lambda i: (0, 0)),
          pl.BlockSpec((NH, 1), lambda i: (0, 0)),
          pl.BlockSpec((1, 1), lambda i: (0, 0)),
      ],
      out_specs=pl.BlockSpec((G, 1), lambda i: (0, 0)),
      out_shape=jax.ShapeDtypeStruct((G, 1), jnp.float32),
      scratch_shapes=[pltpu.VMEM((G, NH), jnp.float32)],
  )(acc, hws, dis, batchp, gcn_b.reshape(1, NH), f1w, f1b.reshape(1, NH),
    f2w, f2b.reshape(1, 1))


@jax.jit
def kernel(x, edge_index, batch, known_mask, unk_mask, obs_mask, msg_weights,
           emb_W, emb_b, gcn_W, gcn_b, fc1_W, fc1_b, fc2_W, fc2_b):
  sp, wc, bc = _prep_call(msg_weights, emb_W, emb_b, gcn_W)
  sp16 = jnp.pad(sp.reshape(3), (0, 13))

  sentinel = jnp.full((MPAD - known_mask.shape[0],), EPAD, jnp.int32)
  km = jnp.concatenate([known_mask, sentinel])
  um = jnp.concatenate([unk_mask, sentinel])
  om = jnp.concatenate([obs_mask, sentinel])
  ew = _ew_call(km, um, om, sp16)

  zpad = jnp.zeros((EPAD - E,), jnp.int32)
  row = jnp.concatenate([edge_index[0], zpad])
  col = jnp.concatenate([edge_index[1], zpad])
  degp = _deg_call(col, ew)

  xpad = jnp.pad(x, ((0, NPAD - N), (0, 0)))
  hw = _mm_call(xpad, wc, bc)
  hws, dis = _scale_call(degp[0].reshape(NPAD, 1), degp[1].reshape(NPAD, 1),
                         hw)
  acc = _edge_call(row, col, ew, hws)

  batchp = jnp.pad(batch, (0, NPAD - N), constant_values=G).reshape(NPAD, 1)
  return _final_call(acc, hws, dis, batchp, gcn_b, fc1_W, fc1_b, fc2_W, fc2_b)
